# baseline (speedup 1.0000x reference)
import functools

import jax
import jax.numpy as jnp
from jax import lax
from jax.experimental import pallas as pl
from jax.experimental.pallas import tpu as pltpu
from jax.experimental.pallas import tpu_sc as plsc

N = 5000
NP = 5120
P = 1024
NCP = 256
HID = 64
TK = 64
IOU_THR = 0.7
RT1 = 512
RT = 1024
CT = 1024
NRT = NP // RT
NCT = NP // CT
SEG = 128
NSEGB = NP // SEG
NEG = float("-inf")
BIG = 2**30


def _dot(a, b):
    return lax.dot_general(a, b, (((1,), (0,)), ((), ())),
                           preferred_element_type=jnp.float32)


def _dot_nt(a, b):
    return lax.dot_general(a, b, (((1,), (1,)), ((), ())),
                           preferred_element_type=jnp.float32)


def _proj_body(cls_ref, f_ref, w1s_ref, b1s_ref, w2s_ref, b2s_ref,
               w1o_ref, b1o_ref, w2o_ref, b2o_ref, s_ref, o_ref):
    x = cls_ref[...]
    f = f_ref[...]

    def mlp(w1, b1, w2, b2):
        h = jnp.maximum(_dot(x, w1[...]) + b1[...], 0.0)
        return _dot(h, w2[...]) + b2[...]

    s_ref[...] = mlp(w1s_ref, b1s_ref, w2s_ref, b2s_ref) * f
    o_ref[...] = mlp(w1o_ref, b1o_ref, w2o_ref, b2o_ref) * f


def _score_body(s_ref, o_ref, lm_ref, m_ref, macc):
    i = pl.program_id(0)
    j = pl.program_id(1)
    logit = lax.cond(
        i > j,
        lambda _: jnp.zeros((RT, CT), jnp.float32),
        lambda _: _dot_nt(s_ref[...], o_ref[...]),
        0,
    )
    row = i * RT + lax.broadcasted_iota(jnp.int32, (RT, CT), 0)
    col = j * CT + lax.broadcasted_iota(jnp.int32, (RT, CT), 1)
    v = jnp.where(col > row, logit, 0.0)
    v = jnp.where((col == row) | (row >= N) | (col >= N), NEG, v)
    lm_ref[...] = v
    segmax = jnp.max(v, axis=1, keepdims=True)

    @pl.when(j == 0)
    def _():
        macc[...] = segmax

    @pl.when(j > 0)
    def _():
        macc[...] = jnp.maximum(macc[...], segmax)

    @pl.when(j == NCT - 1)
    def _():
        m_ref[...] = macc[...]


def _topk_body(m_ref, lm_hbm, vals_ref, si2_ref, oi2_ref, sc_ref,
               b_ref, sem):
    flatiota = (lax.broadcasted_iota(jnp.int32, (40, 128), 0) * 128
                + lax.broadcasted_iota(jnp.int32, (40, 128), 1))
    iota64c = lax.broadcasted_iota(jnp.int32, (TK, 1), 0)
    lane128 = lax.broadcasted_iota(jnp.int32, (1, SEG), 1)

    def rowsel(k, carry):
        m, rowids = carry
        mval = jnp.max(m)
        r = jnp.min(jnp.where(m == mval, flatiota, BIG))
        m = jnp.where(flatiota == r, NEG, m)
        rowids = jnp.where(iota64c == k, r, rowids)
        pltpu.make_async_copy(
            lm_hbm.at[pl.ds(r, 1), :], b_ref.at[pl.ds(k, 1), :], sem
        ).start()
        return m, rowids

    m0 = m_ref[...]
    rowids0 = jnp.zeros((TK, 1), jnp.int32)
    _, rowids = lax.fori_loop(0, TK, rowsel, (m0, rowids0))

    def roww(k, _):
        rk = jnp.min(jnp.where(iota64c == k, rowids, BIG))
        pltpu.make_async_copy(
            lm_hbm.at[pl.ds(rk, 1), :], b_ref.at[pl.ds(k, 1), :], sem
        ).wait()
        return 0

    lax.fori_loop(0, TK, roww, 0)

    siota = (lax.broadcasted_iota(jnp.int32, (TK, NSEGB), 0) * NSEGB
             + lax.broadcasted_iota(jnp.int32, (TK, NSEGB), 1))
    sm = jnp.full((TK, NSEGB), NEG, jnp.float32)
    laneseg = lax.broadcasted_iota(jnp.int32, (TK, NSEGB), 1)
    for c in range(NSEGB):
        segmax = jnp.max(b_ref[:, c * SEG:(c + 1) * SEG], axis=1,
                         keepdims=True)
        sm = jnp.where(laneseg == c, segmax, sm)
    sc_ref[...] = sm

    def extract(k, carry):
        vrow, si2, oi2 = carry
        s = sc_ref[...]
        val = jnp.max(s)
        hits = s == val
        cnt = jnp.sum(hits.astype(jnp.int32))
        sflat = jnp.min(jnp.where(hits, siota, BIG))

        def fast(_):
            brow = sflat // NSEGB
            c = sflat % NSEGB
            base = (brow // 8) * 8
            rsel = lax.broadcasted_iota(jnp.int32, (8, 1), 0) == brow - base
            seg8 = b_ref[pl.ds(base, 8), pl.ds(c * SEG, SEG)]
            seg = jnp.max(jnp.where(rsel, seg8, NEG), axis=0, keepdims=True)
            jloc = jnp.min(jnp.where(seg == val, lane128, BIG))
            oi = c * SEG + jloc
            rt = jnp.min(jnp.where(iota64c == brow, rowids, BIG))
            return rt, oi, brow

        def slow(_):
            flatb = (rowids * N
                     + lax.broadcasted_iota(jnp.int32, (TK, NP), 1))
            bb = b_ref[...]
            fm = jnp.min(jnp.where(bb == val, flatb, BIG))
            rt = fm // N
            oi = fm % N
            brow = jnp.min(jnp.where(rowids == rt, iota64c, BIG))
            return rt, oi, brow

        rt, oi, brow = lax.cond(cnt > 1, slow, fast, 0)
        c = oi // SEG
        jloc = oi - c * SEG
        base = (brow // 8) * 8
        rsel = lax.broadcasted_iota(jnp.int32, (8, 1), 0) == brow - base
        seg8 = b_ref[pl.ds(base, 8), pl.ds(c * SEG, SEG)]
        seg8 = jnp.where(rsel & (lane128 == jloc), NEG, seg8)
        b_ref[pl.ds(base, 8), pl.ds(c * SEG, SEG)] = seg8
        nm = jnp.max(jnp.where(rsel, seg8, NEG))
        sc_ref[...] = jnp.where(siota == brow * NSEGB + c, nm, sc_ref[...])
        vrow = jnp.where(iota64c == k, val, vrow)
        si2 = jnp.where(iota64c == k, rt, si2)
        oi2 = jnp.where(iota64c == k, oi, oi2)
        return vrow, si2, oi2

    vrow0 = jnp.zeros((TK, 1), jnp.float32)
    idx0 = jnp.zeros((TK, 1), jnp.int32)
    vrow, si2, oi2 = lax.fori_loop(0, TK, extract, (vrow0, idx0, idx0))

    vals_ref[...] = 1.0 / (1.0 + jnp.exp(-vrow))
    si2_ref[...] = si2
    oi2_ref[...] = oi2


def _nms_body(si_ref, oi_ref, boxes_ref, pb_ref, keep_ref):
    si = si_ref[...]
    oi = oi_ref[...]
    boxes = boxes_ref[...]
    col_np = lax.broadcasted_iota(jnp.int32, (TK, NP), 1)
    ohs = (col_np == si).astype(jnp.float32)
    oho = (col_np == oi).astype(jnp.float32)
    bs = lax.dot_general(ohs, boxes, (((1,), (0,)), ((), ())),
                         precision=lax.Precision.HIGHEST,
                         preferred_element_type=jnp.float32)
    bo = lax.dot_general(oho, boxes, (((1,), (0,)), ((), ())),
                         precision=lax.Precision.HIGHEST,
                         preferred_element_type=jnp.float32)
    row_np = lax.broadcasted_iota(jnp.int32, (NP, TK), 0)
    ohsT = (row_np == jnp.transpose(si)).astype(jnp.float32)
    ohoT = (row_np == jnp.transpose(oi)).astype(jnp.float32)
    bsT = lax.dot_general(boxes, ohsT, (((0,), (0,)), ((), ())),
                          precision=lax.Precision.HIGHEST,
                          preferred_element_type=jnp.float32)
    boT = lax.dot_general(boxes, ohoT, (((0,), (0,)), ((), ())),
                          precision=lax.Precision.HIGHEST,
                          preferred_element_type=jnp.float32)

    ux1 = jnp.minimum(bs[:, 0:1], bo[:, 0:1])
    uy1 = jnp.minimum(bs[:, 1:2], bo[:, 1:2])
    ux2 = jnp.maximum(bs[:, 2:3], bo[:, 2:3])
    uy2 = jnp.maximum(bs[:, 3:4], bo[:, 3:4])
    ux1T = jnp.minimum(bsT[0:1, :], boT[0:1, :])
    uy1T = jnp.minimum(bsT[1:2, :], boT[1:2, :])
    ux2T = jnp.maximum(bsT[2:3, :], boT[2:3, :])
    uy2T = jnp.maximum(bsT[3:4, :], boT[3:4, :])

    x1 = jnp.maximum(ux1, ux1T)
    y1 = jnp.maximum(uy1, uy1T)
    x2 = jnp.minimum(ux2, ux2T)
    y2 = jnp.minimum(uy2, uy2T)
    inter = jnp.maximum(x2 - x1, 0.0) * jnp.maximum(y2 - y1, 0.0)
    area = jnp.maximum(ux2 - ux1, 0.0) * jnp.maximum(uy2 - uy1, 0.0)
    areaT = jnp.maximum(ux2T - ux1T, 0.0) * jnp.maximum(uy2T - uy1T, 0.0)
    union = area + areaT - inter
    iou = inter / jnp.maximum(union, 1e-9)

    row64 = lax.broadcasted_iota(jnp.int32, (TK, TK), 0)
    lane64 = lax.broadcasted_iota(jnp.int32, (1, TK), 1)

    def step(i, keep):
        rowv = jnp.max(jnp.where(row64 == i, iou, NEG), axis=0,
                       keepdims=True)
        act = (lane64 < i) & (keep > 0.5) & (rowv > IOU_THR)
        sup = jnp.max(act.astype(jnp.float32))
        return jnp.where(lane64 == i, 1.0 - sup, keep)

    keep = lax.fori_loop(1, TK, step, jnp.ones((1, TK), jnp.float32))
    keep_ref[...] = keep
    pb_ref[...] = jnp.concatenate([bs, bo], axis=1)


@functools.lru_cache(maxsize=1)
def _make_sc_gather():
    mesh = plsc.VectorSubcoreMesh(core_axis_name="c", subcore_axis_name="s")
    npairs_w = 8
    nw_used = TK // npairs_w

    @functools.partial(
        pl.kernel, mesh=mesh,
        out_type=(jax.ShapeDtypeStruct((TK, P), jnp.float32),
                  jax.ShapeDtypeStruct((TK, P), jnp.float32),
                  jax.ShapeDtypeStruct((TK, P), jnp.float32)),
        scratch_types=[pltpu.VMEM((npairs_w,), jnp.int32),
                       pltpu.VMEM((npairs_w,), jnp.int32),
                       pltpu.VMEM((npairs_w, P), jnp.float32),
                       pltpu.VMEM((npairs_w, P), jnp.float32),
                       pltpu.VMEM((npairs_w, P), jnp.float32),
                       pltpu.SemaphoreType.DMA],
    )
    def sc_gather(feat_hbm, si_hbm, oi_hbm, fs_hbm, fo_hbm, fm_hbm,
                  si_v, oi_v, fs_v, fo_v, fm_v, sem):
        info = plsc.get_sparse_core_info()
        wid = lax.axis_index("s") * info.num_cores + lax.axis_index("c")

        @pl.when(wid < nw_used)
        def _():
            base = wid * npairs_w
            pltpu.sync_copy(si_hbm.at[pl.ds(base, npairs_w)], si_v)
            pltpu.sync_copy(oi_hbm.at[pl.ds(base, npairs_w)], oi_v)
            pltpu.async_copy(feat_hbm.at[si_v], fs_v, sem).wait()
            pltpu.async_copy(feat_hbm.at[oi_v], fo_v, sem).wait()
            for r in range(npairs_w):
                for i in range(P // 16):
                    a = fs_v[r, pl.ds(i * 16, 16)]
                    b = fo_v[r, pl.ds(i * 16, 16)]
                    fm_v[r, pl.ds(i * 16, 16)] = (a + b) * 0.5
            pltpu.sync_copy(fs_v, fs_hbm.at[pl.ds(base, npairs_w)])
            pltpu.sync_copy(fo_v, fo_hbm.at[pl.ds(base, npairs_w)])
            pltpu.sync_copy(fm_v, fm_hbm.at[pl.ds(base, npairs_w)])

    return sc_gather


def _sc_gather(features, si, oi):
    return _make_sc_gather()(features, si, oi)


def kernel(scores, features, boxes, W1s, b1s, W2s, b2s, W1o, b1o, W2o, b2o):
    f32 = jnp.float32
    cls_p = jnp.zeros((NP, NCP), f32).at[:N, :151].set(scores[:, :151])
    feat_p = jnp.zeros((NP, P), f32).at[:N].set(features)
    boxes_p = jnp.zeros((NP, 4), f32).at[:N].set(boxes)
    w1s_p = jnp.zeros((NCP, HID), f32).at[:151].set(W1s)
    w1o_p = jnp.zeros((NCP, HID), f32).at[:151].set(W1o)
    b1s_p = b1s.reshape(1, HID)
    b1o_p = b1o.reshape(1, HID)
    b2s_p = b2s.reshape(1, P)
    b2o_p = b2o.reshape(1, P)

    s, o = pl.pallas_call(
        _proj_body,
        grid=(NP // RT1,),
        in_specs=[
            pl.BlockSpec((RT1, NCP), lambda i: (i, 0)),
            pl.BlockSpec((RT1, P), lambda i: (i, 0)),
            pl.BlockSpec((NCP, HID), lambda i: (0, 0)),
            pl.BlockSpec((1, HID), lambda i: (0, 0)),
            pl.BlockSpec((HID, P), lambda i: (0, 0)),
            pl.BlockSpec((1, P), lambda i: (0, 0)),
            pl.BlockSpec((NCP, HID), lambda i: (0, 0)),
            pl.BlockSpec((1, HID), lambda i: (0, 0)),
            pl.BlockSpec((HID, P), lambda i: (0, 0)),
            pl.BlockSpec((1, P), lambda i: (0, 0)),
        ],
        out_specs=[pl.BlockSpec((RT1, P), lambda i: (i, 0)),
                   pl.BlockSpec((RT1, P), lambda i: (i, 0))],
        out_shape=[jax.ShapeDtypeStruct((NP, P), f32),
                   jax.ShapeDtypeStruct((NP, P), f32)],
    )(cls_p, feat_p, w1s_p, b1s_p, W2s, b2s_p, w1o_p, b1o_p, W2o, b2o_p)

    lm, m = pl.pallas_call(
        _score_body,
        grid=(NRT, NCT),
        in_specs=[pl.BlockSpec((RT, P), lambda i, j: (i, 0)),
                  pl.BlockSpec((CT, P),
                               lambda i, j: (jnp.maximum(i, j), 0))],
        out_specs=[pl.BlockSpec((RT, CT), lambda i, j: (i, j)),
                   pl.BlockSpec((RT, 1), lambda i, j: (i, 0))],
        out_shape=[jax.ShapeDtypeStruct((NP, NP), f32),
                   jax.ShapeDtypeStruct((NP, 1), f32)],
        scratch_shapes=[pltpu.VMEM((RT, 1), f32)],
        compiler_params=pltpu.CompilerParams(
            dimension_semantics=("arbitrary", "arbitrary")),
    )(s, o)

    m2 = m.reshape(40, 128)

    vals2, si2, oi2 = pl.pallas_call(
        _topk_body,
        in_specs=[pl.BlockSpec(memory_space=pltpu.VMEM),
                  pl.BlockSpec(memory_space=pl.ANY)],
        out_specs=[pl.BlockSpec(memory_space=pltpu.VMEM)] * 3,
        out_shape=[jax.ShapeDtypeStruct((TK, 1), f32),
                   jax.ShapeDtypeStruct((TK, 1), jnp.int32),
                   jax.ShapeDtypeStruct((TK, 1), jnp.int32)],
        scratch_shapes=[pltpu.VMEM((TK, NSEGB), f32),
                        pltpu.VMEM((TK, NP), f32),
                        pltpu.SemaphoreType.DMA],
    )(m2, lm)

    pb, keep = pl.pallas_call(
        _nms_body,
        in_specs=[pl.BlockSpec(memory_space=pltpu.VMEM)] * 3,
        out_specs=[pl.BlockSpec(memory_space=pltpu.VMEM)] * 2,
        out_shape=[jax.ShapeDtypeStruct((TK, 8), f32),
                   jax.ShapeDtypeStruct((1, TK), f32)],
    )(si2, oi2, boxes_p)

    fs, fo, fm = _sc_gather(features, si2.reshape(TK), oi2.reshape(TK))

    pair_boxes = pb.reshape(TK, 2, 4)
    pair_feats = jnp.stack([fs, fo, fm], axis=1)
    vals = vals2.reshape(TK)
    keep_f = keep.reshape(TK)
    return pair_boxes, pair_feats, vals, keep_f


# speedup vs baseline: 1.0498x; 1.0498x over previous
import functools

import jax
import jax.numpy as jnp
from jax import lax
from jax.experimental import pallas as pl
from jax.experimental.pallas import tpu as pltpu
from jax.experimental.pallas import tpu_sc as plsc

N = 5000
NP = 5120
P = 1024
NCP = 152
C_IN = 152
HID = 64
TK = 64
IOU_THR = 0.7
RT1 = 512
RT = 1024
CT = 1024
NRT = NP // RT
NCT = NP // CT
SEG = 128
NSEGB = NP // SEG
NEG = float("-inf")
BIG = 2**30


def _dot(a, b):
    return lax.dot_general(a, b, (((1,), (0,)), ((), ())),
                           preferred_element_type=jnp.float32)


def _dot_nt(a, b):
    return lax.dot_general(a, b, (((1,), (1,)), ((), ())),
                           preferred_element_type=jnp.float32)


def _proj_body(cls_ref, f_ref, w1s_ref, b1s_ref, w2s_ref, b2s_ref,
               w1o_ref, b1o_ref, w2o_ref, b2o_ref, s_ref, o_ref):
    x = cls_ref[...]
    f = f_ref[...]

    def mlp(w1, b1, w2, b2):
        h = jnp.maximum(_dot(x, w1[...]) + b1[...], 0.0)
        return _dot(h, w2[...]) + b2[...]

    s_ref[...] = mlp(w1s_ref, b1s_ref, w2s_ref, b2s_ref) * f
    o_ref[...] = mlp(w1o_ref, b1o_ref, w2o_ref, b2o_ref) * f


def _score_body(s_ref, o_ref, m_ref, macc):
    i = pl.program_id(0)
    j = pl.program_id(1)
    logit = lax.cond(
        i > j,
        lambda _: jnp.zeros((RT, CT), jnp.float32),
        lambda _: _dot_nt(s_ref[...], o_ref[...]),
        0,
    )
    row = i * RT + lax.broadcasted_iota(jnp.int32, (RT, CT), 0)
    col = j * CT + lax.broadcasted_iota(jnp.int32, (RT, CT), 1)
    v = jnp.where(col > row, logit, 0.0)
    v = jnp.where((col == row) | (row >= N) | (col >= N), NEG, v)
    segmax = jnp.max(v, axis=1, keepdims=True)

    @pl.when(j == 0)
    def _():
        macc[...] = segmax

    @pl.when(j > 0)
    def _():
        macc[...] = jnp.maximum(macc[...], segmax)

    @pl.when(j == NCT - 1)
    def _():
        m_ref[...] = macc[...]


def _topk_body(m_ref, s_hbm, o_ref, vals_ref, si2_ref, oi2_ref, sc_ref,
               b_ref, ssel_ref, sem):
    flatiota = (lax.broadcasted_iota(jnp.int32, (40, 128), 0) * 128
                + lax.broadcasted_iota(jnp.int32, (40, 128), 1))
    iota64c = lax.broadcasted_iota(jnp.int32, (TK, 1), 0)
    lane128 = lax.broadcasted_iota(jnp.int32, (1, SEG), 1)

    def rowsel(k, carry):
        m, rowids = carry
        mval = jnp.max(m)
        r = jnp.min(jnp.where(m == mval, flatiota, BIG))
        m = jnp.where(flatiota == r, NEG, m)
        rowids = jnp.where(iota64c == k, r, rowids)
        pltpu.make_async_copy(
            s_hbm.at[pl.ds(r, 1), :], ssel_ref.at[pl.ds(k, 1), :], sem
        ).start()
        return m, rowids

    m0 = m_ref[...]
    rowids0 = jnp.zeros((TK, 1), jnp.int32)
    _, rowids = lax.fori_loop(0, TK, rowsel, (m0, rowids0))

    def roww(k, _):
        rk = jnp.min(jnp.where(iota64c == k, rowids, BIG))
        pltpu.make_async_copy(
            s_hbm.at[pl.ds(rk, 1), :], ssel_ref.at[pl.ds(k, 1), :], sem
        ).wait()
        return 0

    lax.fori_loop(0, TK, roww, 0)

    logits = _dot_nt(ssel_ref[...], o_ref[...])
    colb = lax.broadcasted_iota(jnp.int32, (TK, NP), 1)
    vb = jnp.where(colb > rowids, logits, 0.0)
    vb = jnp.where((colb == rowids) | (colb >= N), NEG, vb)
    b_ref[...] = vb

    siota = (lax.broadcasted_iota(jnp.int32, (TK, NSEGB), 0) * NSEGB
             + lax.broadcasted_iota(jnp.int32, (TK, NSEGB), 1))
    sm = jnp.full((TK, NSEGB), NEG, jnp.float32)
    laneseg = lax.broadcasted_iota(jnp.int32, (TK, NSEGB), 1)
    for c in range(NSEGB):
        segmax = jnp.max(b_ref[:, c * SEG:(c + 1) * SEG], axis=1,
                         keepdims=True)
        sm = jnp.where(laneseg == c, segmax, sm)
    sc_ref[...] = sm

    def extract(k, carry):
        vrow, si2, oi2 = carry
        s = sc_ref[...]
        val = jnp.max(s)
        hits = s == val
        cnt = jnp.sum(hits.astype(jnp.int32))
        sflat = jnp.min(jnp.where(hits, siota, BIG))

        def fast(_):
            brow = sflat // NSEGB
            c = sflat % NSEGB
            base = (brow // 8) * 8
            rsel = lax.broadcasted_iota(jnp.int32, (8, 1), 0) == brow - base
            seg8 = b_ref[pl.ds(base, 8), pl.ds(c * SEG, SEG)]
            seg = jnp.max(jnp.where(rsel, seg8, NEG), axis=0, keepdims=True)
            jloc = jnp.min(jnp.where(seg == val, lane128, BIG))
            oi = c * SEG + jloc
            rt = jnp.min(jnp.where(iota64c == brow, rowids, BIG))
            return rt, oi, brow

        def slow(_):
            flatb = (rowids * N
                     + lax.broadcasted_iota(jnp.int32, (TK, NP), 1))
            bb = b_ref[...]
            fm = jnp.min(jnp.where(bb == val, flatb, BIG))
            rt = fm // N
            oi = fm % N
            brow = jnp.min(jnp.where(rowids == rt, iota64c, BIG))
            return rt, oi, brow

        rt, oi, brow = lax.cond(cnt > 1, slow, fast, 0)
        c = oi // SEG
        jloc = oi - c * SEG
        base = (brow // 8) * 8
        rsel = lax.broadcasted_iota(jnp.int32, (8, 1), 0) == brow - base
        seg8 = b_ref[pl.ds(base, 8), pl.ds(c * SEG, SEG)]
        seg8 = jnp.where(rsel & (lane128 == jloc), NEG, seg8)
        b_ref[pl.ds(base, 8), pl.ds(c * SEG, SEG)] = seg8
        nm = jnp.max(jnp.where(rsel, seg8, NEG))
        sc_ref[...] = jnp.where(siota == brow * NSEGB + c, nm, sc_ref[...])
        vrow = jnp.where(iota64c == k, val, vrow)
        si2 = jnp.where(iota64c == k, rt, si2)
        oi2 = jnp.where(iota64c == k, oi, oi2)
        return vrow, si2, oi2

    vrow0 = jnp.zeros((TK, 1), jnp.float32)
    idx0 = jnp.zeros((TK, 1), jnp.int32)
    vrow, si2, oi2 = lax.fori_loop(0, TK, extract, (vrow0, idx0, idx0))

    vals_ref[...] = 1.0 / (1.0 + jnp.exp(-vrow))
    si2_ref[...] = si2
    oi2_ref[...] = oi2


def _nms_body(si_ref, oi_ref, boxes_ref, pb_ref, keep_ref):
    si = si_ref[...]
    oi = oi_ref[...]
    boxes = boxes_ref[...]
    col_np = lax.broadcasted_iota(jnp.int32, (TK, NP), 1)
    ohs = (col_np == si).astype(jnp.float32)
    oho = (col_np == oi).astype(jnp.float32)
    bs = lax.dot_general(ohs, boxes, (((1,), (0,)), ((), ())),
                         precision=lax.Precision.HIGHEST,
                         preferred_element_type=jnp.float32)
    bo = lax.dot_general(oho, boxes, (((1,), (0,)), ((), ())),
                         precision=lax.Precision.HIGHEST,
                         preferred_element_type=jnp.float32)
    row_np = lax.broadcasted_iota(jnp.int32, (NP, TK), 0)
    ohsT = (row_np == jnp.transpose(si)).astype(jnp.float32)
    ohoT = (row_np == jnp.transpose(oi)).astype(jnp.float32)
    bsT = lax.dot_general(boxes, ohsT, (((0,), (0,)), ((), ())),
                          precision=lax.Precision.HIGHEST,
                          preferred_element_type=jnp.float32)
    boT = lax.dot_general(boxes, ohoT, (((0,), (0,)), ((), ())),
                          precision=lax.Precision.HIGHEST,
                          preferred_element_type=jnp.float32)

    ux1 = jnp.minimum(bs[:, 0:1], bo[:, 0:1])
    uy1 = jnp.minimum(bs[:, 1:2], bo[:, 1:2])
    ux2 = jnp.maximum(bs[:, 2:3], bo[:, 2:3])
    uy2 = jnp.maximum(bs[:, 3:4], bo[:, 3:4])
    ux1T = jnp.minimum(bsT[0:1, :], boT[0:1, :])
    uy1T = jnp.minimum(bsT[1:2, :], boT[1:2, :])
    ux2T = jnp.maximum(bsT[2:3, :], boT[2:3, :])
    uy2T = jnp.maximum(bsT[3:4, :], boT[3:4, :])

    x1 = jnp.maximum(ux1, ux1T)
    y1 = jnp.maximum(uy1, uy1T)
    x2 = jnp.minimum(ux2, ux2T)
    y2 = jnp.minimum(uy2, uy2T)
    inter = jnp.maximum(x2 - x1, 0.0) * jnp.maximum(y2 - y1, 0.0)
    area = jnp.maximum(ux2 - ux1, 0.0) * jnp.maximum(uy2 - uy1, 0.0)
    areaT = jnp.maximum(ux2T - ux1T, 0.0) * jnp.maximum(uy2T - uy1T, 0.0)
    union = area + areaT - inter
    iou = inter / jnp.maximum(union, 1e-9)

    row64 = lax.broadcasted_iota(jnp.int32, (TK, TK), 0)
    lane64 = lax.broadcasted_iota(jnp.int32, (1, TK), 1)

    def step(i, keep):
        rowv = jnp.max(jnp.where(row64 == i, iou, NEG), axis=0,
                       keepdims=True)
        act = (lane64 < i) & (keep > 0.5) & (rowv > IOU_THR)
        sup = jnp.max(act.astype(jnp.float32))
        return jnp.where(lane64 == i, 1.0 - sup, keep)

    keep = lax.fori_loop(1, TK, step, jnp.ones((1, TK), jnp.float32))
    keep_ref[...] = keep
    pb_ref[...] = jnp.concatenate([bs, bo], axis=1)


@functools.lru_cache(maxsize=1)
def _make_sc_gather():
    mesh = plsc.VectorSubcoreMesh(core_axis_name="c", subcore_axis_name="s")
    npairs_w = 8
    nw_used = TK // npairs_w

    @functools.partial(
        pl.kernel, mesh=mesh,
        out_type=(jax.ShapeDtypeStruct((TK, P), jnp.float32),
                  jax.ShapeDtypeStruct((TK, P), jnp.float32),
                  jax.ShapeDtypeStruct((TK, P), jnp.float32)),
        scratch_types=[pltpu.VMEM((npairs_w,), jnp.int32),
                       pltpu.VMEM((npairs_w,), jnp.int32),
                       pltpu.VMEM((npairs_w, P), jnp.float32),
                       pltpu.VMEM((npairs_w, P), jnp.float32),
                       pltpu.VMEM((npairs_w, P), jnp.float32),
                       pltpu.SemaphoreType.DMA],
    )
    def sc_gather(feat_hbm, si_hbm, oi_hbm, fs_hbm, fo_hbm, fm_hbm,
                  si_v, oi_v, fs_v, fo_v, fm_v, sem):
        info = plsc.get_sparse_core_info()
        wid = lax.axis_index("s") * info.num_cores + lax.axis_index("c")

        @pl.when(wid < nw_used)
        def _():
            base = wid * npairs_w
            pltpu.sync_copy(si_hbm.at[pl.ds(base, npairs_w)], si_v)
            pltpu.sync_copy(oi_hbm.at[pl.ds(base, npairs_w)], oi_v)
            pltpu.async_copy(feat_hbm.at[si_v], fs_v, sem).wait()
            pltpu.async_copy(feat_hbm.at[oi_v], fo_v, sem).wait()
            for r in range(npairs_w):
                for i in range(P // 16):
                    a = fs_v[r, pl.ds(i * 16, 16)]
                    b = fo_v[r, pl.ds(i * 16, 16)]
                    fm_v[r, pl.ds(i * 16, 16)] = (a + b) * 0.5
            pltpu.sync_copy(fs_v, fs_hbm.at[pl.ds(base, npairs_w)])
            pltpu.sync_copy(fo_v, fo_hbm.at[pl.ds(base, npairs_w)])
            pltpu.sync_copy(fm_v, fm_hbm.at[pl.ds(base, npairs_w)])

    return sc_gather


def _sc_gather(features, si, oi):
    return _make_sc_gather()(features, si, oi)


def kernel(scores, features, boxes, W1s, b1s, W2s, b2s, W1o, b1o, W2o, b2o):
    f32 = jnp.float32
    boxes_p = jnp.zeros((NP, 4), f32).at[:N].set(boxes)
    w1s_p = jnp.zeros((NCP, HID), f32).at[:151].set(W1s)
    w1o_p = jnp.zeros((NCP, HID), f32).at[:151].set(W1o)
    b1s_p = b1s.reshape(1, HID)
    b1o_p = b1o.reshape(1, HID)
    b2s_p = b2s.reshape(1, P)
    b2o_p = b2o.reshape(1, P)

    s, o = pl.pallas_call(
        _proj_body,
        grid=(NP // RT1,),
        in_specs=[
            pl.BlockSpec((RT1, C_IN), lambda i: (i, 0)),
            pl.BlockSpec((RT1, P), lambda i: (i, 0)),
            pl.BlockSpec((NCP, HID), lambda i: (0, 0)),
            pl.BlockSpec((1, HID), lambda i: (0, 0)),
            pl.BlockSpec((HID, P), lambda i: (0, 0)),
            pl.BlockSpec((1, P), lambda i: (0, 0)),
            pl.BlockSpec((NCP, HID), lambda i: (0, 0)),
            pl.BlockSpec((1, HID), lambda i: (0, 0)),
            pl.BlockSpec((HID, P), lambda i: (0, 0)),
            pl.BlockSpec((1, P), lambda i: (0, 0)),
        ],
        out_specs=[pl.BlockSpec((RT1, P), lambda i: (i, 0)),
                   pl.BlockSpec((RT1, P), lambda i: (i, 0))],
        out_shape=[jax.ShapeDtypeStruct((NP, P), f32),
                   jax.ShapeDtypeStruct((NP, P), f32)],
    )(scores, features, w1s_p, b1s_p, W2s, b2s_p, w1o_p, b1o_p, W2o, b2o_p)

    m = pl.pallas_call(
        _score_body,
        grid=(NRT, NCT),
        in_specs=[pl.BlockSpec((RT, P), lambda i, j: (i, 0)),
                  pl.BlockSpec((CT, P),
                               lambda i, j: (jnp.maximum(i, j), 0))],
        out_specs=pl.BlockSpec((RT, 1), lambda i, j: (i, 0)),
        out_shape=jax.ShapeDtypeStruct((NP, 1), f32),
        scratch_shapes=[pltpu.VMEM((RT, 1), f32)],
        compiler_params=pltpu.CompilerParams(
            dimension_semantics=("arbitrary", "arbitrary")),
    )(s, o)

    m2 = m.reshape(40, 128)

    vals2, si2, oi2 = pl.pallas_call(
        _topk_body,
        in_specs=[pl.BlockSpec(memory_space=pltpu.VMEM),
                  pl.BlockSpec(memory_space=pl.ANY),
                  pl.BlockSpec(memory_space=pltpu.VMEM)],
        out_specs=[pl.BlockSpec(memory_space=pltpu.VMEM)] * 3,
        out_shape=[jax.ShapeDtypeStruct((TK, 1), f32),
                   jax.ShapeDtypeStruct((TK, 1), jnp.int32),
                   jax.ShapeDtypeStruct((TK, 1), jnp.int32)],
        scratch_shapes=[pltpu.VMEM((TK, NSEGB), f32),
                        pltpu.VMEM((TK, NP), f32),
                        pltpu.VMEM((TK, P), f32),
                        pltpu.SemaphoreType.DMA],
    )(m2, s, o)

    pb, keep = pl.pallas_call(
        _nms_body,
        in_specs=[pl.BlockSpec(memory_space=pltpu.VMEM)] * 3,
        out_specs=[pl.BlockSpec(memory_space=pltpu.VMEM)] * 2,
        out_shape=[jax.ShapeDtypeStruct((TK, 8), f32),
                   jax.ShapeDtypeStruct((1, TK), f32)],
    )(si2, oi2, boxes_p)

    fs, fo, fm = _sc_gather(features, si2.reshape(TK), oi2.reshape(TK))

    pair_boxes = pb.reshape(TK, 2, 4)
    pair_feats = jnp.stack([fs, fo, fm], axis=1)
    vals = vals2.reshape(TK)
    keep_f = keep.reshape(TK)
    return pair_boxes, pair_feats, vals, keep_f


# speedup vs baseline: 1.0577x; 1.0075x over previous
import functools

import jax
import jax.numpy as jnp
from jax import lax
from jax.experimental import pallas as pl
from jax.experimental.pallas import tpu as pltpu
from jax.experimental.pallas import tpu_sc as plsc

N = 5000
NP = 5120
P = 1024
NCP = 152
C_IN = 152
HID = 64
TK = 64
IOU_THR = 0.7
RT1 = 512
RT = 1024
CT = 1024
NRT = NP // RT
NCT = NP // CT
SEG = 128
NSEGB = NP // SEG
NEG = float("-inf")
BIG = 2**30


def _dot(a, b):
    return lax.dot_general(a, b, (((1,), (0,)), ((), ())),
                           preferred_element_type=jnp.float32)


def _dot_nt(a, b):
    return lax.dot_general(a, b, (((1,), (1,)), ((), ())),
                           preferred_element_type=jnp.float32)


def _proj_body(cls_ref, f_ref, w1s_ref, b1s_ref, w2s_ref, b2s_ref,
               w1o_ref, b1o_ref, w2o_ref, b2o_ref, s_ref, o_ref):
    x = cls_ref[...]
    f = f_ref[...]

    def mlp(w1, b1, w2, b2):
        h = jnp.maximum(_dot(x, w1[...]) + b1[...], 0.0)
        return _dot(h, w2[...]) + b2[...]

    s_ref[...] = mlp(w1s_ref, b1s_ref, w2s_ref, b2s_ref) * f
    o_ref[...] = mlp(w1o_ref, b1o_ref, w2o_ref, b2o_ref) * f


def _score_body(s_ref, o_ref, m_ref, macc):
    i = pl.program_id(0)
    j = pl.program_id(1)
    logit = lax.cond(
        i > j,
        lambda _: jnp.zeros((RT, CT), jnp.float32),
        lambda _: _dot_nt(s_ref[...], o_ref[...]),
        0,
    )
    row = i * RT + lax.broadcasted_iota(jnp.int32, (RT, CT), 0)
    col = j * CT + lax.broadcasted_iota(jnp.int32, (RT, CT), 1)
    v = jnp.where(col > row, logit, 0.0)
    v = jnp.where((col == row) | (row >= N) | (col >= N), NEG, v)
    segmax = jnp.max(v, axis=1, keepdims=True)

    @pl.when(j == 0)
    def _():
        macc[...] = segmax

    @pl.when(j > 0)
    def _():
        macc[...] = jnp.maximum(macc[...], segmax)

    @pl.when(j == NCT - 1)
    def _():
        m_ref[...] = macc[...]


def _amax2d(x):
    return jnp.max(jnp.max(x, axis=1, keepdims=True), axis=0, keepdims=True)


def _amin2d(x):
    return jnp.min(jnp.min(x, axis=1, keepdims=True), axis=0, keepdims=True)


def _topk_body(m_ref, s_hbm, o_hbm, vals_ref, si2_ref, oi2_ref, sc_ref,
               b_ref, ssel_ref, s_ref, o_ref, sem_s, sem_o):
    cp_s = pltpu.make_async_copy(s_hbm, s_ref, sem_s)
    cp_o = pltpu.make_async_copy(o_hbm, o_ref, sem_o)
    cp_s.start()
    cp_o.start()

    flatiota = (lax.broadcasted_iota(jnp.int32, (40, 128), 0) * 128
                + lax.broadcasted_iota(jnp.int32, (40, 128), 1))
    iota64c = lax.broadcasted_iota(jnp.int32, (TK, 1), 0)
    iota8 = lax.broadcasted_iota(jnp.int32, (8, 1), 0)
    lane128 = lax.broadcasted_iota(jnp.int32, (1, SEG), 1)

    def rowsel(k, carry):
        m, rowids = carry
        mval = _amax2d(m)
        r = _amin2d(jnp.where(m == mval, flatiota, BIG))
        m = jnp.where(flatiota == r, NEG, m)
        rowids = jnp.where(iota64c == k, r, rowids)
        return m, rowids

    m0 = m_ref[...]
    rowids0 = jnp.zeros((TK, 1), jnp.int32)
    _, rowids = lax.fori_loop(0, TK, rowsel, (m0, rowids0))

    cp_s.wait()

    def gath(k, ssel):
        r = jnp.min(jnp.where(iota64c == k, rowids, BIG))
        base = (r // 8) * 8
        s8 = s_ref[pl.ds(base, 8), :]
        row = jnp.sum(jnp.where(iota8 == r - base, s8, 0.0), axis=0,
                      keepdims=True)
        return jnp.where(iota64c == k, row, ssel)

    ssel = lax.fori_loop(0, TK, gath, jnp.zeros((TK, P), jnp.float32))
    ssel_ref[...] = ssel

    cp_o.wait()
    logits = _dot_nt(ssel_ref[...], o_ref[...])
    colb = lax.broadcasted_iota(jnp.int32, (TK, NP), 1)
    vb = jnp.where(colb > rowids, logits, 0.0)
    vb = jnp.where((colb == rowids) | (colb >= N), NEG, vb)
    b_ref[...] = vb

    siota = (lax.broadcasted_iota(jnp.int32, (TK, NSEGB), 0) * NSEGB
             + lax.broadcasted_iota(jnp.int32, (TK, NSEGB), 1))
    sm = jnp.full((TK, NSEGB), NEG, jnp.float32)
    laneseg = lax.broadcasted_iota(jnp.int32, (TK, NSEGB), 1)
    for c in range(NSEGB):
        segmax = jnp.max(b_ref[:, c * SEG:(c + 1) * SEG], axis=1,
                         keepdims=True)
        sm = jnp.where(laneseg == c, segmax, sm)
    sc_ref[...] = sm

    def extract(k, carry):
        vrow, si2, oi2 = carry
        s = sc_ref[...]
        val = _amax2d(s)
        hits = s == val
        rmin = _amin2d(jnp.where(hits, rowids, BIG))
        brow = jnp.min(jnp.where(rowids == rmin, iota64c, BIG))
        c = jnp.min(jnp.where(hits & (rowids == rmin), laneseg, BIG))
        base = (brow // 8) * 8
        rsel = iota8 == brow - base
        seg8 = b_ref[pl.ds(base, 8), pl.ds(c * SEG, SEG)]
        segrow = jnp.max(jnp.where(rsel, seg8, NEG), axis=0, keepdims=True)
        jloc = jnp.min(jnp.where(segrow == val, lane128, BIG),
                       axis=1, keepdims=True)
        seg8 = jnp.where(rsel & (lane128 == jloc), NEG, seg8)
        b_ref[pl.ds(base, 8), pl.ds(c * SEG, SEG)] = seg8
        nm = _amax2d(jnp.where(rsel, seg8, NEG))
        sc_ref[...] = jnp.where(siota == brow * NSEGB + c, nm, sc_ref[...])
        vrow = jnp.where(iota64c == k, val, vrow)
        si2 = jnp.where(iota64c == k, rmin, si2)
        oi2 = jnp.where(iota64c == k, c * SEG + jloc, oi2)
        return vrow, si2, oi2

    vrow0 = jnp.zeros((TK, 1), jnp.float32)
    idx0 = jnp.zeros((TK, 1), jnp.int32)
    vrow, si2, oi2 = lax.fori_loop(0, TK, extract, (vrow0, idx0, idx0))

    vals_ref[...] = 1.0 / (1.0 + jnp.exp(-vrow))
    si2_ref[...] = si2
    oi2_ref[...] = oi2


def _nms_body(si_ref, oi_ref, boxes_ref, pb_ref, keep_ref):
    si = si_ref[...]
    oi = oi_ref[...]
    boxes = boxes_ref[...]
    col_np = lax.broadcasted_iota(jnp.int32, (TK, NP), 1)
    ohs = (col_np == si).astype(jnp.float32)
    oho = (col_np == oi).astype(jnp.float32)
    bs = lax.dot_general(ohs, boxes, (((1,), (0,)), ((), ())),
                         precision=lax.Precision.HIGHEST,
                         preferred_element_type=jnp.float32)
    bo = lax.dot_general(oho, boxes, (((1,), (0,)), ((), ())),
                         precision=lax.Precision.HIGHEST,
                         preferred_element_type=jnp.float32)
    row_np = lax.broadcasted_iota(jnp.int32, (NP, TK), 0)
    ohsT = (row_np == jnp.transpose(si)).astype(jnp.float32)
    ohoT = (row_np == jnp.transpose(oi)).astype(jnp.float32)
    bsT = lax.dot_general(boxes, ohsT, (((0,), (0,)), ((), ())),
                          precision=lax.Precision.HIGHEST,
                          preferred_element_type=jnp.float32)
    boT = lax.dot_general(boxes, ohoT, (((0,), (0,)), ((), ())),
                          precision=lax.Precision.HIGHEST,
                          preferred_element_type=jnp.float32)

    ux1 = jnp.minimum(bs[:, 0:1], bo[:, 0:1])
    uy1 = jnp.minimum(bs[:, 1:2], bo[:, 1:2])
    ux2 = jnp.maximum(bs[:, 2:3], bo[:, 2:3])
    uy2 = jnp.maximum(bs[:, 3:4], bo[:, 3:4])
    ux1T = jnp.minimum(bsT[0:1, :], boT[0:1, :])
    uy1T = jnp.minimum(bsT[1:2, :], boT[1:2, :])
    ux2T = jnp.maximum(bsT[2:3, :], boT[2:3, :])
    uy2T = jnp.maximum(bsT[3:4, :], boT[3:4, :])

    x1 = jnp.maximum(ux1, ux1T)
    y1 = jnp.maximum(uy1, uy1T)
    x2 = jnp.minimum(ux2, ux2T)
    y2 = jnp.minimum(uy2, uy2T)
    inter = jnp.maximum(x2 - x1, 0.0) * jnp.maximum(y2 - y1, 0.0)
    area = jnp.maximum(ux2 - ux1, 0.0) * jnp.maximum(uy2 - uy1, 0.0)
    areaT = jnp.maximum(ux2T - ux1T, 0.0) * jnp.maximum(uy2T - uy1T, 0.0)
    union = area + areaT - inter
    iou = inter / jnp.maximum(union, 1e-9)

    row64 = lax.broadcasted_iota(jnp.int32, (TK, TK), 0)
    lane64 = lax.broadcasted_iota(jnp.int32, (1, TK), 1)

    def step(i, keep):
        rowv = jnp.max(jnp.where(row64 == i, iou, NEG), axis=0,
                       keepdims=True)
        act = (lane64 < i) & (keep > 0.5) & (rowv > IOU_THR)
        sup = jnp.max(act.astype(jnp.float32))
        return jnp.where(lane64 == i, 1.0 - sup, keep)

    keep = lax.fori_loop(1, TK, step, jnp.ones((1, TK), jnp.float32))
    keep_ref[...] = keep
    pb_ref[...] = jnp.concatenate([bs, bo], axis=1)


@functools.lru_cache(maxsize=1)
def _make_sc_gather():
    mesh = plsc.VectorSubcoreMesh(core_axis_name="c", subcore_axis_name="s")
    npairs_w = 8
    nw_used = TK // npairs_w

    @functools.partial(
        pl.kernel, mesh=mesh,
        out_type=(jax.ShapeDtypeStruct((TK, P), jnp.float32),
                  jax.ShapeDtypeStruct((TK, P), jnp.float32),
                  jax.ShapeDtypeStruct((TK, P), jnp.float32)),
        scratch_types=[pltpu.VMEM((npairs_w,), jnp.int32),
                       pltpu.VMEM((npairs_w,), jnp.int32),
                       pltpu.VMEM((npairs_w, P), jnp.float32),
                       pltpu.VMEM((npairs_w, P), jnp.float32),
                       pltpu.VMEM((npairs_w, P), jnp.float32),
                       pltpu.SemaphoreType.DMA],
    )
    def sc_gather(feat_hbm, si_hbm, oi_hbm, fs_hbm, fo_hbm, fm_hbm,
                  si_v, oi_v, fs_v, fo_v, fm_v, sem):
        info = plsc.get_sparse_core_info()
        wid = lax.axis_index("s") * info.num_cores + lax.axis_index("c")

        @pl.when(wid < nw_used)
        def _():
            base = wid * npairs_w
            pltpu.sync_copy(si_hbm.at[pl.ds(base, npairs_w)], si_v)
            pltpu.sync_copy(oi_hbm.at[pl.ds(base, npairs_w)], oi_v)
            pltpu.async_copy(feat_hbm.at[si_v], fs_v, sem).wait()
            pltpu.async_copy(feat_hbm.at[oi_v], fo_v, sem).wait()
            for r in range(npairs_w):
                for i in range(P // 16):
                    a = fs_v[r, pl.ds(i * 16, 16)]
                    b = fo_v[r, pl.ds(i * 16, 16)]
                    fm_v[r, pl.ds(i * 16, 16)] = (a + b) * 0.5
            pltpu.sync_copy(fs_v, fs_hbm.at[pl.ds(base, npairs_w)])
            pltpu.sync_copy(fo_v, fo_hbm.at[pl.ds(base, npairs_w)])
            pltpu.sync_copy(fm_v, fm_hbm.at[pl.ds(base, npairs_w)])

    return sc_gather


def _sc_gather(features, si, oi):
    return _make_sc_gather()(features, si, oi)


def kernel(scores, features, boxes, W1s, b1s, W2s, b2s, W1o, b1o, W2o, b2o):
    f32 = jnp.float32
    boxes_p = jnp.zeros((NP, 4), f32).at[:N].set(boxes)
    w1s_p = jnp.zeros((NCP, HID), f32).at[:151].set(W1s)
    w1o_p = jnp.zeros((NCP, HID), f32).at[:151].set(W1o)
    b1s_p = b1s.reshape(1, HID)
    b1o_p = b1o.reshape(1, HID)
    b2s_p = b2s.reshape(1, P)
    b2o_p = b2o.reshape(1, P)

    s, o = pl.pallas_call(
        _proj_body,
        grid=(NP // RT1,),
        in_specs=[
            pl.BlockSpec((RT1, C_IN), lambda i: (i, 0)),
            pl.BlockSpec((RT1, P), lambda i: (i, 0)),
            pl.BlockSpec((NCP, HID), lambda i: (0, 0)),
            pl.BlockSpec((1, HID), lambda i: (0, 0)),
            pl.BlockSpec((HID, P), lambda i: (0, 0)),
            pl.BlockSpec((1, P), lambda i: (0, 0)),
            pl.BlockSpec((NCP, HID), lambda i: (0, 0)),
            pl.BlockSpec((1, HID), lambda i: (0, 0)),
            pl.BlockSpec((HID, P), lambda i: (0, 0)),
            pl.BlockSpec((1, P), lambda i: (0, 0)),
        ],
        out_specs=[pl.BlockSpec((RT1, P), lambda i: (i, 0)),
                   pl.BlockSpec((RT1, P), lambda i: (i, 0))],
        out_shape=[jax.ShapeDtypeStruct((NP, P), f32),
                   jax.ShapeDtypeStruct((NP, P), f32)],
    )(scores, features, w1s_p, b1s_p, W2s, b2s_p, w1o_p, b1o_p, W2o, b2o_p)

    m = pl.pallas_call(
        _score_body,
        grid=(NRT, NCT),
        in_specs=[pl.BlockSpec((RT, P), lambda i, j: (i, 0)),
                  pl.BlockSpec((CT, P),
                               lambda i, j: (jnp.maximum(i, j), 0))],
        out_specs=pl.BlockSpec((RT, 1), lambda i, j: (i, 0)),
        out_shape=jax.ShapeDtypeStruct((NP, 1), f32),
        scratch_shapes=[pltpu.VMEM((RT, 1), f32)],
        compiler_params=pltpu.CompilerParams(
            dimension_semantics=("arbitrary", "arbitrary")),
    )(s, o)

    m2 = m.reshape(40, 128)

    vals2, si2, oi2 = pl.pallas_call(
        _topk_body,
        in_specs=[pl.BlockSpec(memory_space=pltpu.VMEM),
                  pl.BlockSpec(memory_space=pl.ANY),
                  pl.BlockSpec(memory_space=pl.ANY)],
        out_specs=[pl.BlockSpec(memory_space=pltpu.VMEM)] * 3,
        out_shape=[jax.ShapeDtypeStruct((TK, 1), f32),
                   jax.ShapeDtypeStruct((TK, 1), jnp.int32),
                   jax.ShapeDtypeStruct((TK, 1), jnp.int32)],
        scratch_shapes=[pltpu.VMEM((TK, NSEGB), f32),
                        pltpu.VMEM((TK, NP), f32),
                        pltpu.VMEM((TK, P), f32),
                        pltpu.VMEM((NP, P), f32),
                        pltpu.VMEM((NP, P), f32),
                        pltpu.SemaphoreType.DMA,
                        pltpu.SemaphoreType.DMA],
    )(m2, s, o)

    pb, keep = pl.pallas_call(
        _nms_body,
        in_specs=[pl.BlockSpec(memory_space=pltpu.VMEM)] * 3,
        out_specs=[pl.BlockSpec(memory_space=pltpu.VMEM)] * 2,
        out_shape=[jax.ShapeDtypeStruct((TK, 8), f32),
                   jax.ShapeDtypeStruct((1, TK), f32)],
    )(si2, oi2, boxes_p)

    fs, fo, fm = _sc_gather(features, si2.reshape(TK), oi2.reshape(TK))

    pair_boxes = pb.reshape(TK, 2, 4)
    pair_feats = jnp.stack([fs, fo, fm], axis=1)
    vals = vals2.reshape(TK)
    keep_f = keep.reshape(TK)
    return pair_boxes, pair_feats, vals, keep_f


# speedup vs baseline: 1.1829x; 1.1184x over previous
import functools

import jax
import jax.numpy as jnp
from jax import lax
from jax.experimental import pallas as pl
from jax.experimental.pallas import tpu as pltpu
from jax.experimental.pallas import tpu_sc as plsc

N = 5000
NP = 5120
P = 1024
NCP = 152
C_IN = 152
HID = 64
TK = 64
IOU_THR = 0.7
RT1 = 1024
RT = 1024
CT = 1024
NRT = NP // RT
NCT = NP // CT
SEG = 128
NSEGB = NP // SEG
NEG = float("-inf")
BIG = 2**30


def _dot(a, b):
    return lax.dot_general(a, b, (((1,), (0,)), ((), ())),
                           preferred_element_type=jnp.float32)


def _dot_nt(a, b):
    return lax.dot_general(a, b, (((1,), (1,)), ((), ())),
                           preferred_element_type=jnp.float32)


def _proj_body(cls_ref, f_ref, w1s_ref, b1s_ref, w2s_ref, b2s_ref,
               w1o_ref, b1o_ref, w2o_ref, b2o_ref, s_ref, o_ref):
    x = cls_ref[...]
    f = f_ref[...]

    def mlp(w1, b1, w2, b2):
        h = jnp.maximum(_dot(x, w1[...]) + b1[...], 0.0)
        return _dot(h, w2[...]) + b2[...]

    s = mlp(w1s_ref, b1s_ref, w2s_ref, b2s_ref) * f
    o = mlp(w1o_ref, b1o_ref, w2o_ref, b2o_ref) * f
    i = pl.program_id(0)
    row = i * RT1 + lax.broadcasted_iota(jnp.int32, (RT1, 1), 0)

    @pl.when(i == NP // RT1 - 1)
    def _():
        s_ref[...] = jnp.where(row < N, s, 0.0)
        o_ref[...] = jnp.where(row < N, o, 0.0)

    @pl.when(i < NP // RT1 - 1)
    def _():
        s_ref[...] = s
        o_ref[...] = o


def _score_body(s_ref, o_ref, m_ref, macc):
    i = pl.program_id(0)
    j = pl.program_id(1)
    row1 = i * RT + lax.broadcasted_iota(jnp.int32, (RT, 1), 0)

    def seg_lower(_):
        return jnp.where(row1 >= N, NEG, jnp.float32(0.0))

    def seg_diag(_):
        logit = _dot_nt(s_ref[...], o_ref[...])
        row = i * RT + lax.broadcasted_iota(jnp.int32, (RT, CT), 0)
        col = j * CT + lax.broadcasted_iota(jnp.int32, (RT, CT), 1)
        v = jnp.where(col > row, logit, 0.0)
        v = jnp.where((col == row) | (row >= N) | (col >= N), NEG, v)
        return jnp.max(v, axis=1, keepdims=True)

    def seg_upper(_):
        logit = _dot_nt(s_ref[...], o_ref[...])

        def pad(_):
            col = j * CT + lax.broadcasted_iota(jnp.int32, (RT, CT), 1)
            return jnp.max(jnp.where(col >= N, NEG, logit), axis=1,
                           keepdims=True)

        def pure(_):
            return jnp.max(logit, axis=1, keepdims=True)

        return lax.cond(j == NCT - 1, pad, pure, 0)

    segmax = lax.cond(
        i > j, seg_lower,
        lambda u: lax.cond(i == j, seg_diag, seg_upper, u), 0)

    @pl.when(j == 0)
    def _():
        macc[...] = segmax

    @pl.when(j > 0)
    def _():
        macc[...] = jnp.maximum(macc[...], segmax)

    @pl.when(j == NCT - 1)
    def _():
        m_ref[...] = macc[...]


def _amax2d(x):
    return jnp.max(jnp.max(x, axis=1, keepdims=True), axis=0, keepdims=True)


def _amin2d(x):
    return jnp.min(jnp.min(x, axis=1, keepdims=True), axis=0, keepdims=True)


def _topk_body(m_ref, s_hbm, o_hbm, vals_ref, si2_ref, oi2_ref, sc_ref,
               b_ref, ssel_ref, s_ref, o_ref, sem_s, sem_o):
    cp_s = pltpu.make_async_copy(s_hbm, s_ref, sem_s)
    cp_o = pltpu.make_async_copy(o_hbm, o_ref, sem_o)
    cp_s.start()
    cp_o.start()

    flatiota = (lax.broadcasted_iota(jnp.int32, (40, 128), 0) * 128
                + lax.broadcasted_iota(jnp.int32, (40, 128), 1))
    iota64c = lax.broadcasted_iota(jnp.int32, (TK, 1), 0)
    iota8 = lax.broadcasted_iota(jnp.int32, (8, 1), 0)
    lane128 = lax.broadcasted_iota(jnp.int32, (1, SEG), 1)

    def rowsel(k, carry):
        m, rowids = carry
        mval = _amax2d(m)
        r = _amin2d(jnp.where(m == mval, flatiota, BIG))
        m = jnp.where(flatiota == r, NEG, m)
        rowids = jnp.where(iota64c == k, r, rowids)
        return m, rowids

    m0 = m_ref[...]
    rowids0 = jnp.zeros((TK, 1), jnp.int32)
    _, rowids = lax.fori_loop(0, TK, rowsel, (m0, rowids0))

    cp_s.wait()
    colg = lax.broadcasted_iota(jnp.int32, (TK, NP), 1)
    ohr = (colg == rowids).astype(jnp.float32)
    ssel_ref[...] = lax.dot_general(
        ohr, s_ref[...], (((1,), (0,)), ((), ())),
        precision=lax.Precision.HIGHEST, preferred_element_type=jnp.float32)

    cp_o.wait()
    logits = _dot_nt(ssel_ref[...], o_ref[...])
    colb = lax.broadcasted_iota(jnp.int32, (TK, NP), 1)
    vb = jnp.where(colb > rowids, logits, 0.0)
    vb = jnp.where((colb == rowids) | (colb >= N), NEG, vb)
    b_ref[...] = vb

    siota = (lax.broadcasted_iota(jnp.int32, (TK, NSEGB), 0) * NSEGB
             + lax.broadcasted_iota(jnp.int32, (TK, NSEGB), 1))
    sm = jnp.full((TK, NSEGB), NEG, jnp.float32)
    laneseg = lax.broadcasted_iota(jnp.int32, (TK, NSEGB), 1)
    for c in range(NSEGB):
        segmax = jnp.max(b_ref[:, c * SEG:(c + 1) * SEG], axis=1,
                         keepdims=True)
        sm = jnp.where(laneseg == c, segmax, sm)
    sc_ref[...] = sm

    def extract(k, carry):
        vrow, si2, oi2 = carry
        s = sc_ref[...]
        val = _amax2d(s)
        hits = s == val
        rmin = _amin2d(jnp.where(hits, rowids, BIG))
        brow = jnp.min(jnp.where(rowids == rmin, iota64c, BIG))
        c = jnp.min(jnp.where(hits & (rowids == rmin), laneseg, BIG))
        base = (brow // 8) * 8
        rsel = iota8 == brow - base
        seg8 = b_ref[pl.ds(base, 8), pl.ds(c * SEG, SEG)]
        segrow = jnp.max(jnp.where(rsel, seg8, NEG), axis=0, keepdims=True)
        jloc = jnp.min(jnp.where(segrow == val, lane128, BIG),
                       axis=1, keepdims=True)
        seg8 = jnp.where(rsel & (lane128 == jloc), NEG, seg8)
        b_ref[pl.ds(base, 8), pl.ds(c * SEG, SEG)] = seg8
        nm = _amax2d(jnp.where(rsel, seg8, NEG))
        sc_ref[...] = jnp.where(siota == brow * NSEGB + c, nm, sc_ref[...])
        vrow = jnp.where(iota64c == k, val, vrow)
        si2 = jnp.where(iota64c == k, rmin, si2)
        oi2 = jnp.where(iota64c == k, c * SEG + jloc, oi2)
        return vrow, si2, oi2

    vrow0 = jnp.zeros((TK, 1), jnp.float32)
    idx0 = jnp.zeros((TK, 1), jnp.int32)
    vrow, si2, oi2 = lax.fori_loop(0, TK, extract, (vrow0, idx0, idx0))

    vals_ref[...] = 1.0 / (1.0 + jnp.exp(-vrow))
    si2_ref[...] = si2
    oi2_ref[...] = oi2


def _nms_body(si_ref, oi_ref, boxes_ref, pb_ref, keep_ref):
    si = si_ref[...]
    oi = oi_ref[...]
    boxes = boxes_ref[...]
    col_np = lax.broadcasted_iota(jnp.int32, (TK, NP), 1)
    ohs = (col_np == si).astype(jnp.float32)
    oho = (col_np == oi).astype(jnp.float32)
    bs = lax.dot_general(ohs, boxes, (((1,), (0,)), ((), ())),
                         precision=lax.Precision.HIGHEST,
                         preferred_element_type=jnp.float32)
    bo = lax.dot_general(oho, boxes, (((1,), (0,)), ((), ())),
                         precision=lax.Precision.HIGHEST,
                         preferred_element_type=jnp.float32)
    row_np = lax.broadcasted_iota(jnp.int32, (NP, TK), 0)
    ohsT = (row_np == jnp.transpose(si)).astype(jnp.float32)
    ohoT = (row_np == jnp.transpose(oi)).astype(jnp.float32)
    bsT = lax.dot_general(boxes, ohsT, (((0,), (0,)), ((), ())),
                          precision=lax.Precision.HIGHEST,
                          preferred_element_type=jnp.float32)
    boT = lax.dot_general(boxes, ohoT, (((0,), (0,)), ((), ())),
                          precision=lax.Precision.HIGHEST,
                          preferred_element_type=jnp.float32)

    ux1 = jnp.minimum(bs[:, 0:1], bo[:, 0:1])
    uy1 = jnp.minimum(bs[:, 1:2], bo[:, 1:2])
    ux2 = jnp.maximum(bs[:, 2:3], bo[:, 2:3])
    uy2 = jnp.maximum(bs[:, 3:4], bo[:, 3:4])
    ux1T = jnp.minimum(bsT[0:1, :], boT[0:1, :])
    uy1T = jnp.minimum(bsT[1:2, :], boT[1:2, :])
    ux2T = jnp.maximum(bsT[2:3, :], boT[2:3, :])
    uy2T = jnp.maximum(bsT[3:4, :], boT[3:4, :])

    x1 = jnp.maximum(ux1, ux1T)
    y1 = jnp.maximum(uy1, uy1T)
    x2 = jnp.minimum(ux2, ux2T)
    y2 = jnp.minimum(uy2, uy2T)
    inter = jnp.maximum(x2 - x1, 0.0) * jnp.maximum(y2 - y1, 0.0)
    area = jnp.maximum(ux2 - ux1, 0.0) * jnp.maximum(uy2 - uy1, 0.0)
    areaT = jnp.maximum(ux2T - ux1T, 0.0) * jnp.maximum(uy2T - uy1T, 0.0)
    union = area + areaT - inter
    iou = inter / jnp.maximum(union, 1e-9)

    row64 = lax.broadcasted_iota(jnp.int32, (TK, TK), 0)
    lane64 = lax.broadcasted_iota(jnp.int32, (1, TK), 1)

    def step(i, keep):
        rowv = jnp.max(jnp.where(row64 == i, iou, NEG), axis=0,
                       keepdims=True)
        act = (lane64 < i) & (keep > 0.5) & (rowv > IOU_THR)
        sup = jnp.max(act.astype(jnp.float32))
        return jnp.where(lane64 == i, 1.0 - sup, keep)

    keep = lax.fori_loop(1, TK, step, jnp.ones((1, TK), jnp.float32))
    keep_ref[...] = keep
    pb_ref[...] = jnp.concatenate([bs, bo], axis=1)


@functools.lru_cache(maxsize=1)
def _make_sc_gather():
    mesh = plsc.VectorSubcoreMesh(core_axis_name="c", subcore_axis_name="s")
    npairs_w = 8
    nw_used = TK // npairs_w

    @functools.partial(
        pl.kernel, mesh=mesh,
        out_type=(jax.ShapeDtypeStruct((TK, P), jnp.float32),
                  jax.ShapeDtypeStruct((TK, P), jnp.float32),
                  jax.ShapeDtypeStruct((TK, P), jnp.float32)),
        scratch_types=[pltpu.VMEM((npairs_w,), jnp.int32),
                       pltpu.VMEM((npairs_w,), jnp.int32),
                       pltpu.VMEM((npairs_w, P), jnp.float32),
                       pltpu.VMEM((npairs_w, P), jnp.float32),
                       pltpu.VMEM((npairs_w, P), jnp.float32),
                       pltpu.SemaphoreType.DMA],
    )
    def sc_gather(feat_hbm, si_hbm, oi_hbm, fs_hbm, fo_hbm, fm_hbm,
                  si_v, oi_v, fs_v, fo_v, fm_v, sem):
        info = plsc.get_sparse_core_info()
        wid = lax.axis_index("s") * info.num_cores + lax.axis_index("c")

        @pl.when(wid < nw_used)
        def _():
            base = wid * npairs_w
            pltpu.sync_copy(si_hbm.at[pl.ds(base, npairs_w)], si_v)
            pltpu.sync_copy(oi_hbm.at[pl.ds(base, npairs_w)], oi_v)
            pltpu.async_copy(feat_hbm.at[si_v], fs_v, sem).wait()
            pltpu.async_copy(feat_hbm.at[oi_v], fo_v, sem).wait()
            for r in range(npairs_w):
                for i in range(P // 16):
                    a = fs_v[r, pl.ds(i * 16, 16)]
                    b = fo_v[r, pl.ds(i * 16, 16)]
                    fm_v[r, pl.ds(i * 16, 16)] = (a + b) * 0.5
            pltpu.sync_copy(fs_v, fs_hbm.at[pl.ds(base, npairs_w)])
            pltpu.sync_copy(fo_v, fo_hbm.at[pl.ds(base, npairs_w)])
            pltpu.sync_copy(fm_v, fm_hbm.at[pl.ds(base, npairs_w)])

    return sc_gather


def _sc_gather(features, si, oi):
    return _make_sc_gather()(features, si, oi)


def kernel(scores, features, boxes, W1s, b1s, W2s, b2s, W1o, b1o, W2o, b2o):
    f32 = jnp.float32
    boxes_p = jnp.zeros((NP, 4), f32).at[:N].set(boxes)
    w1s_p = jnp.zeros((NCP, HID), f32).at[:151].set(W1s)
    w1o_p = jnp.zeros((NCP, HID), f32).at[:151].set(W1o)
    b1s_p = b1s.reshape(1, HID)
    b1o_p = b1o.reshape(1, HID)
    b2s_p = b2s.reshape(1, P)
    b2o_p = b2o.reshape(1, P)

    s, o = pl.pallas_call(
        _proj_body,
        grid=(NP // RT1,),
        in_specs=[
            pl.BlockSpec((RT1, C_IN), lambda i: (i, 0)),
            pl.BlockSpec((RT1, P), lambda i: (i, 0)),
            pl.BlockSpec((NCP, HID), lambda i: (0, 0)),
            pl.BlockSpec((1, HID), lambda i: (0, 0)),
            pl.BlockSpec((HID, P), lambda i: (0, 0)),
            pl.BlockSpec((1, P), lambda i: (0, 0)),
            pl.BlockSpec((NCP, HID), lambda i: (0, 0)),
            pl.BlockSpec((1, HID), lambda i: (0, 0)),
            pl.BlockSpec((HID, P), lambda i: (0, 0)),
            pl.BlockSpec((1, P), lambda i: (0, 0)),
        ],
        out_specs=[pl.BlockSpec((RT1, P), lambda i: (i, 0)),
                   pl.BlockSpec((RT1, P), lambda i: (i, 0))],
        out_shape=[jax.ShapeDtypeStruct((NP, P), f32),
                   jax.ShapeDtypeStruct((NP, P), f32)],
    )(scores, features, w1s_p, b1s_p, W2s, b2s_p, w1o_p, b1o_p, W2o, b2o_p)

    m = pl.pallas_call(
        _score_body,
        grid=(NRT, NCT),
        in_specs=[pl.BlockSpec((RT, P), lambda i, j: (i, 0)),
                  pl.BlockSpec((CT, P),
                               lambda i, j: (jnp.maximum(i, j), 0))],
        out_specs=pl.BlockSpec((RT, 1), lambda i, j: (i, 0)),
        out_shape=jax.ShapeDtypeStruct((NP, 1), f32),
        scratch_shapes=[pltpu.VMEM((RT, 1), f32)],
        compiler_params=pltpu.CompilerParams(
            dimension_semantics=("arbitrary", "arbitrary")),
    )(s, o)

    m2 = m.reshape(40, 128)

    vals2, si2, oi2 = pl.pallas_call(
        _topk_body,
        in_specs=[pl.BlockSpec(memory_space=pltpu.VMEM),
                  pl.BlockSpec(memory_space=pl.ANY),
                  pl.BlockSpec(memory_space=pl.ANY)],
        out_specs=[pl.BlockSpec(memory_space=pltpu.VMEM)] * 3,
        out_shape=[jax.ShapeDtypeStruct((TK, 1), f32),
                   jax.ShapeDtypeStruct((TK, 1), jnp.int32),
                   jax.ShapeDtypeStruct((TK, 1), jnp.int32)],
        scratch_shapes=[pltpu.VMEM((TK, NSEGB), f32),
                        pltpu.VMEM((TK, NP), f32),
                        pltpu.VMEM((TK, P), f32),
                        pltpu.VMEM((NP, P), f32),
                        pltpu.VMEM((NP, P), f32),
                        pltpu.SemaphoreType.DMA,
                        pltpu.SemaphoreType.DMA],
    )(m2, s, o)

    pb, keep = pl.pallas_call(
        _nms_body,
        in_specs=[pl.BlockSpec(memory_space=pltpu.VMEM)] * 3,
        out_specs=[pl.BlockSpec(memory_space=pltpu.VMEM)] * 2,
        out_shape=[jax.ShapeDtypeStruct((TK, 8), f32),
                   jax.ShapeDtypeStruct((1, TK), f32)],
    )(si2, oi2, boxes_p)

    fs, fo, fm = _sc_gather(features, si2.reshape(TK), oi2.reshape(TK))

    pair_boxes = pb.reshape(TK, 2, 4)
    pair_feats = jnp.stack([fs, fo, fm], axis=1)
    vals = vals2.reshape(TK)
    keep_f = keep.reshape(TK)
    return pair_boxes, pair_feats, vals, keep_f
